# baseline (speedup 1.0000x reference)
import jax
import jax.numpy as jnp
from jax import lax
from jax.experimental import pallas as pl
from jax.experimental.pallas import tpu as pltpu
from jax.experimental.pallas import tpu_sc as plsc

N, P2, W2, CKV, TOPK = 8, 64, 64, 192, 4
NC, NS, LANES = 2, 16, 16
NW = NC * NS
IPW = N * P2 // NW
WPB = P2 // IPW
SPW = IPW * TOPK
GSZ = 2
NGRP = SPW // GSZ
NBUF = 3


def _body(kv_hbm, idx_hbm, out_hbm, idx_v, bufs, gsems, ssems):
    sid = lax.axis_index("s")
    wid = sid * NC + lax.axis_index("c")
    b = wid // WPB
    i0 = (wid % WPB) * IPW

    pltpu.sync_copy(idx_hbm.at[pl.ds(wid * SPW, SPW)], idx_v)

    kv_b = kv_hbm.at[b]
    out_b = out_hbm.at[b]
    my_bufs = bufs.at[sid]

    groups = [idx_v[pl.ds(g * LANES, LANES)] for g in range(SPW // LANES)]

    def slab_index(s):
        return groups[s // LANES][s % LANES]

    def gather_start(g, j):
        s = g * GSZ + j
        return pltpu.async_copy(
            kv_b.at[slab_index(s)], my_bufs.at[g % NBUF].at[j], gsems[g % NBUF]
        )

    def scatter_start(g):
        i = i0 + (g * GSZ) // TOPK
        k = (g * GSZ) % TOPK
        dst = out_b.at[i].at[pl.ds(k, GSZ)]
        return pltpu.async_copy(my_bufs.at[g % NBUF], dst, ssems[g % NBUF])

    gh = [None] * NGRP
    sh = [None] * NGRP
    for step in range(NGRP + 2):
        if step < NGRP:
            if step - NBUF >= 0:
                sh[step - NBUF].wait()
                sh[step - NBUF] = None
            gh[step] = [gather_start(step, j) for j in range(GSZ)]
        if step >= 2:
            g = step - 2
            for h in gh[g]:
                h.wait()
            sh[g] = scatter_start(g)
    for h in sh:
        if h is not None:
            h.wait()


@jax.jit
def kernel(kv, r_idx):
    mesh = plsc.VectorSubcoreMesh(
        core_axis_name="c", subcore_axis_name="s", num_cores=NC, num_subcores=NS
    )
    idx_flat = r_idx.reshape(N * P2 * TOPK)
    return pl.kernel(
        _body,
        out_type=jax.ShapeDtypeStruct((N, P2, TOPK, W2, CKV), jnp.float32),
        mesh=mesh,
        scratch_types=[
            pltpu.VMEM((SPW,), jnp.int32),
            pltpu.VMEM_SHARED((NS, NBUF, GSZ, W2, CKV), jnp.float32),
            [pltpu.SemaphoreType.DMA] * NBUF,
            [pltpu.SemaphoreType.DMA] * NBUF,
        ],
    )(kv, idx_flat)


# speedup vs baseline: 1.0137x; 1.0137x over previous
import jax
import jax.numpy as jnp
from jax import lax
from jax.experimental import pallas as pl
from jax.experimental.pallas import tpu as pltpu
from jax.experimental.pallas import tpu_sc as plsc

N, P2, W2, CKV, TOPK = 8, 64, 64, 192, 4
NC, NS, LANES = 2, 16, 16
NW = NC * NS
IPW = N * P2 // NW
WPB = P2 // IPW
SPW = IPW * TOPK
NBUF = 4
LOOKAHEAD = 2


def _body(kv_hbm, idx_hbm, out_hbm, idx_v, bufs, gsems, ssems):
    sid = lax.axis_index("s")
    wid = sid * NC + lax.axis_index("c")
    b = wid // WPB
    i0 = (wid % WPB) * IPW

    pltpu.sync_copy(idx_hbm.at[pl.ds(wid * SPW, SPW)], idx_v)

    kv_b = kv_hbm.at[b]
    out_b = out_hbm.at[b]
    my_bufs = bufs.at[sid]

    groups = [idx_v[pl.ds(g * LANES, LANES)] for g in range(SPW // LANES)]

    def slab_index(s):
        return groups[s // LANES][s % LANES]

    def gather_start(s):
        return pltpu.async_copy(
            kv_b.at[slab_index(s)], my_bufs.at[s % NBUF], gsems[s % NBUF]
        )

    def scatter_start(s):
        dst = out_b.at[i0 + s // TOPK].at[s % TOPK]
        return pltpu.async_copy(my_bufs.at[s % NBUF], dst, ssems[s % NBUF])

    gh = [None] * SPW
    sh = [None] * SPW
    for s in range(LOOKAHEAD):
        gh[s] = gather_start(s)
    for s in range(SPW):
        if s + LOOKAHEAD < SPW:
            prev = s + LOOKAHEAD - NBUF
            if prev >= 0:
                sh[prev].wait()
                sh[prev] = None
            gh[s + LOOKAHEAD] = gather_start(s + LOOKAHEAD)
        gh[s].wait()
        sh[s] = scatter_start(s)
    for h in sh:
        if h is not None:
            h.wait()


@jax.jit
def kernel(kv, r_idx):
    mesh = plsc.VectorSubcoreMesh(
        core_axis_name="c", subcore_axis_name="s", num_cores=NC, num_subcores=NS
    )
    idx_flat = r_idx.reshape(N * P2 * TOPK)
    return pl.kernel(
        _body,
        out_type=jax.ShapeDtypeStruct((N, P2, TOPK, W2, CKV), jnp.float32),
        mesh=mesh,
        scratch_types=[
            pltpu.VMEM((SPW,), jnp.int32),
            pltpu.VMEM_SHARED((NS, NBUF, W2, CKV), jnp.float32),
            [pltpu.SemaphoreType.DMA] * NBUF,
            [pltpu.SemaphoreType.DMA] * NBUF,
        ],
    )(kv, idx_flat)
